# Initial kernel scaffold; baseline (speedup 1.0000x reference)
import jax
import jax.numpy as jnp
from jax.experimental import pallas as pl


def kernel(x, W, b):
    raise NotImplementedError("write your pallas kernel here")


# speedup vs baseline: 1.0859x; 1.0859x over previous
import jax
import jax.numpy as jnp
from jax.experimental import pallas as pl
from jax.experimental.pallas import tpu as pltpu

_M = 8192
_K = 4096
_N = 4096
_BM = 512
_BN = 1024


def _matmul_kernel(x_ref, w_ref, b_ref, o_ref):
    xb = x_ref[...].astype(jnp.bfloat16)
    wb = w_ref[...].astype(jnp.bfloat16)
    acc = jax.lax.dot_general(
        xb, wb, (((1,), (1,)), ((), ())),
        preferred_element_type=jnp.float32)
    o_ref[...] = acc + b_ref[...]


def kernel(x, W, b):
    x_flat = x.reshape(_M, _K)
    b2 = b.reshape(1, _N)
    grid = (_N // _BN, _M // _BM)
    out = pl.pallas_call(
        _matmul_kernel,
        grid=grid,
        in_specs=[
            pl.BlockSpec((_BM, _K), lambda n, m: (m, 0)),
            pl.BlockSpec((_BN, _K), lambda n, m: (n, 0)),
            pl.BlockSpec((1, _BN), lambda n, m: (0, n)),
        ],
        out_specs=pl.BlockSpec((_BM, _BN), lambda n, m: (m, n)),
        out_shape=jax.ShapeDtypeStruct((_M, _N), jnp.float32),
        compiler_params=pltpu.CompilerParams(
            dimension_semantics=("arbitrary", "arbitrary"),
        ),
    )(x_flat, W, b2)
    return out.reshape(x.shape[0], x.shape[1], _N)
